# baseline (speedup 1.0000x reference)
import functools

import jax
import jax.numpy as jnp
from jax import lax
from jax.experimental import pallas as pl
from jax.experimental.pallas import tpu as pltpu
from jax.experimental.pallas import tpu_sc as plsc

N_TASKS = 10
FILTER_NUM = 128
NHID = 2048
BATCH = 4096

NC = 2
NS = 16
L = 16
NW = NC * NS
B_PER_W = BATCH // NW

TC_BLOCK = 256


def _sigmoid_block(tstage, sv, rows, width):
    def body(r, _):
        for j in range(width // L):
            x = tstage[r, pl.ds(j * L, L)]
            z = sv * x
            ez = jnp.exp(-jnp.abs(z))
            d = 1.0 / (1.0 + ez)
            tstage[r, pl.ds(j * L, L)] = jnp.where(z >= 0, d, ez * d)
        return 0
    lax.fori_loop(0, rows, body, 0)


def _sc_kernel(t_ref, s_ref, ef1,
               of1,
               svec_v, idx_v, tf1, sem):
    cid = lax.axis_index("c")
    sid = lax.axis_index("s")

    pltpu.sync_copy(s_ref, svec_v)
    sv = svec_v[...]

    pltpu.sync_copy(ef1, tf1)
    _sigmoid_block(tf1, sv, N_TASKS, NHID)

    wid = sid * NC + cid
    base = wid * B_PER_W
    pltpu.sync_copy(t_ref.at[pl.ds(base, B_PER_W)], idx_v)

    def chunk_body(c, _):
        v = idx_v[pl.ds(c * L, L)]
        for j in range(L):
            r = v[j]
            i = c * L + j
            pltpu.async_copy(tf1.at[r], of1.at[base + i], sem)
        return 0
    lax.fori_loop(0, B_PER_W // L, chunk_body, 0)

    cp = pltpu.make_async_copy(tf1.at[0], of1.at[base], sem)

    def drain_body(i, _):
        cp.wait()
        return 0
    lax.fori_loop(0, B_PER_W, drain_body, 0)


def _tc_kernel(s_ref, t_ref, tblc1_ref, tblc2_ref, tblc3_ref,
               tbl2_ref, tbl3_ref,
               outc1_ref, outc2_ref, outc3_ref, out2_ref, out3_ref):
    i = pl.program_id(0)
    s = s_ref[0]
    tb = t_ref[pl.ds(i * TC_BLOCK, TC_BLOCK)]
    onehot = (tb[:, None] == lax.broadcasted_iota(jnp.int32, (TC_BLOCK, N_TASKS), 1)
              ).astype(jnp.float32)
    for tbl_ref, out_ref in ((tblc1_ref, outc1_ref), (tblc2_ref, outc2_ref),
                             (tblc3_ref, outc3_ref), (tbl2_ref, out2_ref),
                             (tbl3_ref, out3_ref)):
        sig = jax.nn.sigmoid(s * tbl_ref[...])
        out_ref[...] = jnp.dot(onehot, sig, preferred_element_type=jnp.float32)


@jax.jit
def _run(t, svec, ec1, ec2, ec3, efc1, efc2, efc3):
    f32 = jnp.float32
    sc_call = functools.partial(
        pl.kernel,
        out_type=[
            jax.ShapeDtypeStruct((BATCH, NHID), f32),
        ],
        mesh=plsc.VectorSubcoreMesh(core_axis_name="c", subcore_axis_name="s"),
        scratch_types=[
            pltpu.VMEM((L,), f32),
            pltpu.VMEM((B_PER_W,), jnp.int32),
            pltpu.VMEM((N_TASKS, NHID), f32),
            pltpu.SemaphoreType.DMA,
        ],
    )(_sc_kernel)
    of1, = sc_call(t, svec, efc1)

    oc1, oc2, oc3, of2, of3 = pl.pallas_call(
        _tc_kernel,
        grid=(BATCH // TC_BLOCK,),
        in_specs=[
            pl.BlockSpec(memory_space=pltpu.SMEM),
            pl.BlockSpec(memory_space=pltpu.VMEM),
            pl.BlockSpec(memory_space=pltpu.VMEM),
            pl.BlockSpec(memory_space=pltpu.VMEM),
            pl.BlockSpec(memory_space=pltpu.VMEM),
            pl.BlockSpec(memory_space=pltpu.VMEM),
            pl.BlockSpec(memory_space=pltpu.VMEM),
        ],
        out_specs=[
            pl.BlockSpec((TC_BLOCK, FILTER_NUM), lambda i: (i, 0)),
            pl.BlockSpec((TC_BLOCK, FILTER_NUM), lambda i: (i, 0)),
            pl.BlockSpec((TC_BLOCK, FILTER_NUM), lambda i: (i, 0)),
            pl.BlockSpec((TC_BLOCK, NHID), lambda i: (i, 0)),
            pl.BlockSpec((TC_BLOCK, NHID), lambda i: (i, 0)),
        ],
        out_shape=[
            jax.ShapeDtypeStruct((BATCH, FILTER_NUM), f32),
            jax.ShapeDtypeStruct((BATCH, FILTER_NUM), f32),
            jax.ShapeDtypeStruct((BATCH, FILTER_NUM), f32),
            jax.ShapeDtypeStruct((BATCH, NHID), f32),
            jax.ShapeDtypeStruct((BATCH, NHID), f32),
        ],
    )(svec[:1], t, ec1, ec2, ec3, efc2, efc3)

    return (oc1, oc2, oc3, of1, of2, of3)


def kernel(t, s, ec1, ec2, ec3, efc1, efc2, efc3):
    t32 = t.astype(jnp.int32)
    svec = jnp.full((L,), s, dtype=jnp.float32)
    return _run(t32, svec, ec1, ec2, ec3, efc1, efc2, efc3)


# speedup vs baseline: 1.0539x; 1.0539x over previous
import functools

import jax
import jax.numpy as jnp
from jax import lax
from jax.experimental import pallas as pl
from jax.experimental.pallas import tpu as pltpu
from jax.experimental.pallas import tpu_sc as plsc

N_TASKS = 10
FILTER_NUM = 128
NHID = 2048
BATCH = 4096

NC = 2
NS = 16
L = 16
NW = NC * NS
B_PER_W = BATCH // NW

TC_BLOCK = 256


def _sigmoid_block(tstage, sv, rows, width):
    def body(r, _):
        for j in range(width // L):
            x = tstage[r, pl.ds(j * L, L)]
            z = sv * x
            ez = jnp.exp(-jnp.abs(z))
            d = 1.0 / (1.0 + ez)
            tstage[r, pl.ds(j * L, L)] = jnp.where(z >= 0, d, ez * d)
        return 0
    lax.fori_loop(0, rows, body, 0)


def _sc_kernel(t_ref, s_ref, ef1,
               of1,
               sgf1,
               svec_v, idx_v, tstage, tf1, sem):
    cid = lax.axis_index("c")
    sid = lax.axis_index("s")

    pltpu.sync_copy(s_ref, svec_v)
    sv = svec_v[...]

    col0 = sid * FILTER_NUM
    pltpu.sync_copy(ef1.at[:, pl.ds(col0, FILTER_NUM)], tstage)
    _sigmoid_block(tstage, sv, N_TASKS, FILTER_NUM)
    pltpu.sync_copy(tstage, sgf1.at[:, pl.ds(col0, FILTER_NUM)])
    plsc.subcore_barrier()
    pltpu.sync_copy(sgf1, tf1)

    wid = sid * NC + cid
    base = wid * B_PER_W
    pltpu.sync_copy(t_ref.at[pl.ds(base, B_PER_W)], idx_v)

    def chunk_body(c, _):
        v = idx_v[pl.ds(c * L, L)]
        for j in range(L):
            r = v[j]
            i = c * L + j
            pltpu.async_copy(tf1.at[r], of1.at[base + i], sem)
        return 0
    lax.fori_loop(0, B_PER_W // L, chunk_body, 0)

    cp = pltpu.make_async_copy(tf1.at[0], of1.at[base], sem)

    def drain_body(i, _):
        cp.wait()
        return 0
    lax.fori_loop(0, B_PER_W, drain_body, 0)


def _tc_kernel(s_ref, t_ref, tblc1_ref, tblc2_ref, tblc3_ref,
               tbl2_ref, tbl3_ref,
               outc1_ref, outc2_ref, outc3_ref, out2_ref, out3_ref):
    i = pl.program_id(0)
    s = s_ref[0]
    tb = t_ref[pl.ds(i * TC_BLOCK, TC_BLOCK)]
    onehot = (tb[:, None] == lax.broadcasted_iota(jnp.int32, (TC_BLOCK, N_TASKS), 1)
              ).astype(jnp.float32)
    for tbl_ref, out_ref in ((tblc1_ref, outc1_ref), (tblc2_ref, outc2_ref),
                             (tblc3_ref, outc3_ref), (tbl2_ref, out2_ref),
                             (tbl3_ref, out3_ref)):
        sig = jax.nn.sigmoid(s * tbl_ref[...])
        out_ref[...] = jnp.dot(onehot, sig, preferred_element_type=jnp.float32)


@jax.jit
def _run(t, svec, ec1, ec2, ec3, efc1, efc2, efc3):
    f32 = jnp.float32
    sc_call = functools.partial(
        pl.kernel,
        out_type=[
            jax.ShapeDtypeStruct((BATCH, NHID), f32),
        ],
        mesh=plsc.VectorSubcoreMesh(core_axis_name="c", subcore_axis_name="s"),
        scratch_types=[
            pltpu.HBM((N_TASKS, NHID), f32),
            pltpu.VMEM((L,), f32),
            pltpu.VMEM((B_PER_W,), jnp.int32),
            pltpu.VMEM((N_TASKS, FILTER_NUM), f32),
            pltpu.VMEM((N_TASKS, NHID), f32),
            pltpu.SemaphoreType.DMA,
        ],
    )(_sc_kernel)
    of1, = sc_call(t, svec, efc1)

    oc1, oc2, oc3, of2, of3 = pl.pallas_call(
        _tc_kernel,
        grid=(BATCH // TC_BLOCK,),
        in_specs=[
            pl.BlockSpec(memory_space=pltpu.SMEM),
            pl.BlockSpec(memory_space=pltpu.VMEM),
            pl.BlockSpec(memory_space=pltpu.VMEM),
            pl.BlockSpec(memory_space=pltpu.VMEM),
            pl.BlockSpec(memory_space=pltpu.VMEM),
            pl.BlockSpec(memory_space=pltpu.VMEM),
            pl.BlockSpec(memory_space=pltpu.VMEM),
        ],
        out_specs=[
            pl.BlockSpec((TC_BLOCK, FILTER_NUM), lambda i: (i, 0)),
            pl.BlockSpec((TC_BLOCK, FILTER_NUM), lambda i: (i, 0)),
            pl.BlockSpec((TC_BLOCK, FILTER_NUM), lambda i: (i, 0)),
            pl.BlockSpec((TC_BLOCK, NHID), lambda i: (i, 0)),
            pl.BlockSpec((TC_BLOCK, NHID), lambda i: (i, 0)),
        ],
        out_shape=[
            jax.ShapeDtypeStruct((BATCH, FILTER_NUM), f32),
            jax.ShapeDtypeStruct((BATCH, FILTER_NUM), f32),
            jax.ShapeDtypeStruct((BATCH, FILTER_NUM), f32),
            jax.ShapeDtypeStruct((BATCH, NHID), f32),
            jax.ShapeDtypeStruct((BATCH, NHID), f32),
        ],
    )(svec[:1], t, ec1, ec2, ec3, efc2, efc3)

    return (oc1, oc2, oc3, of1, of2, of3)


def kernel(t, s, ec1, ec2, ec3, efc1, efc2, efc3):
    t32 = t.astype(jnp.int32)
    svec = jnp.full((L,), s, dtype=jnp.float32)
    return _run(t32, svec, ec1, ec2, ec3, efc1, efc2, efc3)


# speedup vs baseline: 1.3116x; 1.2446x over previous
import functools

import jax
import jax.numpy as jnp
from jax import lax
from jax.experimental import pallas as pl
from jax.experimental.pallas import tpu as pltpu
from jax.experimental.pallas import tpu_sc as plsc

N_TASKS = 10
FILTER_NUM = 128
NHID = 2048
BATCH = 4096

NC = 2
NS = 16
L = 16
NW = NC * NS
B_PER_W = BATCH // NW

TC_BLOCK = 256


def _sigmoid_block(tstage, sv, rows, width):
    def body(r, _):
        for j in range(width // L):
            x = tstage[r, pl.ds(j * L, L)]
            z = sv * x
            ez = jnp.exp(-jnp.abs(z))
            d = 1.0 / (1.0 + ez)
            tstage[r, pl.ds(j * L, L)] = jnp.where(z >= 0, d, ez * d)
        return 0
    lax.fori_loop(0, rows, body, 0)


def _sc_kernel(t_ref, s_ref, ec1,
               oc1,
               svec_v, idx_v, tc1, sem):
    cid = lax.axis_index("c")
    sid = lax.axis_index("s")

    pltpu.sync_copy(s_ref, svec_v)
    sv = svec_v[...]

    pltpu.sync_copy(ec1, tc1)
    _sigmoid_block(tc1, sv, N_TASKS, FILTER_NUM)

    wid = sid * NC + cid
    base = wid * B_PER_W
    pltpu.sync_copy(t_ref.at[pl.ds(base, B_PER_W)], idx_v)

    pairs = ((tc1, oc1),)

    def chunk_body(c, _):
        v = idx_v[pl.ds(c * L, L)]
        for j in range(L):
            r = v[j]
            i = c * L + j
            for tbl, out in pairs:
                pltpu.async_copy(tbl.at[r], out.at[base + i], sem)
        return 0
    lax.fori_loop(0, B_PER_W // L, chunk_body, 0)

    cp = pltpu.make_async_copy(tc1.at[0], oc1.at[base], sem)

    def drain_body(i, _):
        cp.wait()
        return 0
    lax.fori_loop(0, len(pairs) * B_PER_W, drain_body, 0)


def _tc_kernel(s_ref, t_ref, tblc2_ref, tblc3_ref, tbl1_ref, tbl2_ref, tbl3_ref,
               outc2_ref, outc3_ref, out1_ref, out2_ref, out3_ref):
    i = pl.program_id(0)
    s = s_ref[0]
    tb = t_ref[pl.ds(i * TC_BLOCK, TC_BLOCK)]
    onehot = (tb[:, None] == lax.broadcasted_iota(jnp.int32, (TC_BLOCK, N_TASKS), 1)
              ).astype(jnp.float32)
    for tbl_ref, out_ref in ((tblc2_ref, outc2_ref), (tblc3_ref, outc3_ref),
                             (tbl1_ref, out1_ref), (tbl2_ref, out2_ref),
                             (tbl3_ref, out3_ref)):
        sig = jax.nn.sigmoid(s * tbl_ref[...])
        out_ref[...] = jnp.dot(onehot, sig, preferred_element_type=jnp.float32)


@jax.jit
def _run(t, svec, ec1, ec2, ec3, efc1, efc2, efc3):
    f32 = jnp.float32
    sc_call = functools.partial(
        pl.kernel,
        out_type=[
            jax.ShapeDtypeStruct((BATCH, FILTER_NUM), f32),
        ],
        mesh=plsc.VectorSubcoreMesh(core_axis_name="c", subcore_axis_name="s"),
        scratch_types=[
            pltpu.VMEM((L,), f32),
            pltpu.VMEM((B_PER_W,), jnp.int32),
            pltpu.VMEM((N_TASKS, FILTER_NUM), f32),
            pltpu.SemaphoreType.DMA,
        ],
    )(_sc_kernel)
    oc1, = sc_call(t, svec, ec1)

    oc2, oc3, of1, of2, of3 = pl.pallas_call(
        _tc_kernel,
        grid=(BATCH // TC_BLOCK,),
        in_specs=[
            pl.BlockSpec(memory_space=pltpu.SMEM),
            pl.BlockSpec(memory_space=pltpu.VMEM),
            pl.BlockSpec(memory_space=pltpu.VMEM),
            pl.BlockSpec(memory_space=pltpu.VMEM),
            pl.BlockSpec(memory_space=pltpu.VMEM),
            pl.BlockSpec(memory_space=pltpu.VMEM),
            pl.BlockSpec(memory_space=pltpu.VMEM),
        ],
        out_specs=[
            pl.BlockSpec((TC_BLOCK, FILTER_NUM), lambda i: (i, 0)),
            pl.BlockSpec((TC_BLOCK, FILTER_NUM), lambda i: (i, 0)),
            pl.BlockSpec((TC_BLOCK, NHID), lambda i: (i, 0)),
            pl.BlockSpec((TC_BLOCK, NHID), lambda i: (i, 0)),
            pl.BlockSpec((TC_BLOCK, NHID), lambda i: (i, 0)),
        ],
        out_shape=[
            jax.ShapeDtypeStruct((BATCH, FILTER_NUM), f32),
            jax.ShapeDtypeStruct((BATCH, FILTER_NUM), f32),
            jax.ShapeDtypeStruct((BATCH, NHID), f32),
            jax.ShapeDtypeStruct((BATCH, NHID), f32),
            jax.ShapeDtypeStruct((BATCH, NHID), f32),
        ],
    )(svec[:1], t, ec2, ec3, efc1, efc2, efc3)

    return (oc1, oc2, oc3, of1, of2, of3)


def kernel(t, s, ec1, ec2, ec3, efc1, efc2, efc3):
    t32 = t.astype(jnp.int32)
    svec = jnp.full((L,), s, dtype=jnp.float32)
    return _run(t32, svec, ec1, ec2, ec3, efc1, efc2, efc3)
